# Initial kernel scaffold; baseline (speedup 1.0000x reference)
import jax
import jax.numpy as jnp
from jax.experimental import pallas as pl


def kernel(xyz, params):
    raise NotImplementedError("write your pallas kernel here")


# speedup vs baseline: 8.9246x; 8.9246x over previous
import functools

import jax
import jax.numpy as jnp
import numpy as np
from jax import lax
from jax.experimental import pallas as pl
from jax.experimental.pallas import tpu as pltpu
from jax.experimental.pallas import tpu_sc as plsc

_BN = 1.0 / np.sqrt(1.0 + 1e-5)
_NPOINTS = [1024, 256, 64]
_K = 32


def _dot(a, b):
    return lax.dot_general(a, b, (((a.ndim - 1,), (0,)), ((), ())),
                           preferred_element_type=jnp.float32)


def _rows_mlp(x, layers):
    layers = [(wt, jnp.zeros((wt.shape[1],), jnp.float32) if b is None else b, a)
              for (wt, b, a) in layers]
    M = x.shape[0]
    Mb = M if M <= 512 else 512
    cout = layers[-1][0].shape[1]

    def body(x_ref, *refs):
        out_ref = refs[-1]
        h = x_ref[...]
        for li in range(len(layers)):
            wt = refs[2 * li][...]
            b = refs[2 * li + 1][...]
            h = _dot(h, wt) + b[None, :]
            if layers[li][2]:
                h = jnp.maximum(h * _BN, 0.0)
        out_ref[...] = h

    in_specs = [pl.BlockSpec((Mb, x.shape[1]), lambda i: (i, 0))]
    args = [x]
    for (wt, b, _a) in layers:
        in_specs.append(pl.BlockSpec(wt.shape, lambda i: (0, 0)))
        in_specs.append(pl.BlockSpec(b.shape, lambda i: (0,)))
        args += [wt, b]
    return pl.pallas_call(
        body,
        grid=(M // Mb,),
        in_specs=in_specs,
        out_specs=pl.BlockSpec((Mb, cout), lambda i: (i, 0)),
        out_shape=jax.ShapeDtypeStruct((M, cout), jnp.float32),
    )(*args)


def _knn_kernel(q, pts, k, with_weights=False):
    B, S, _ = q.shape
    N = pts.shape[1]
    Sb = min(S, 256)

    def body(q_ref, p_ref, *refs):
        if with_weights:
            idx_ref, w_ref, dist_ref, dv_ref = refs
        else:
            idx_ref, dist_ref, dv_ref = refs
        b = pl.program_id(0)
        qb = q_ref[0]
        pb = p_ref[0]
        q2 = jnp.sum(qb * qb, axis=1)
        p2 = jnp.sum(pb * pb, axis=1)
        cross = lax.dot_general(qb, pb, (((1,), (1,)), ((), ())),
                                preferred_element_type=jnp.float32)
        dist_ref[...] = jnp.maximum(q2[:, None] + p2[None, :] - 2.0 * cross, 0.0)
        iota = lax.broadcasted_iota(jnp.int32, (Sb, N), 1)

        kiota = lax.broadcasted_iota(jnp.int32, (Sb, k), 1)

        def step(j, carry):
            d = dist_ref[...]
            m = jnp.min(d, axis=1)
            sel = jnp.min(jnp.where(d <= m[:, None], iota, N), axis=1)
            idx_ref[0] = jnp.where(kiota == j, (sel + b * N)[:, None],
                                   idx_ref[0])
            dist_ref[...] = jnp.where(iota == sel[:, None], jnp.inf, d)
            if with_weights:
                dv_ref[...] = jnp.where(kiota == j, m[:, None], dv_ref[...])
            return carry

        lax.fori_loop(0, k, step, 0)
        if with_weights:
            recip = 1.0 / (dv_ref[...] + 1e-8)
            w_ref[0] = recip / jnp.sum(recip, axis=1, keepdims=True)

    in_specs = [
        pl.BlockSpec((1, Sb, 3), lambda b, s: (b, s, 0)),
        pl.BlockSpec((1, N, 3), lambda b, s: (b, 0, 0)),
    ]
    args = [q, pts]
    out_specs = [pl.BlockSpec((1, Sb, k), lambda b, s: (b, s, 0))]
    out_shape = [jax.ShapeDtypeStruct((B, S, k), jnp.int32)]
    if with_weights:
        out_specs.append(pl.BlockSpec((1, Sb, k), lambda b, s: (b, s, 0)))
        out_shape.append(jax.ShapeDtypeStruct((B, S, k), jnp.float32))
    res = pl.pallas_call(
        body,
        grid=(B, S // Sb),
        in_specs=in_specs,
        out_specs=out_specs,
        out_shape=out_shape,
        scratch_shapes=[pltpu.VMEM((Sb, N), jnp.float32),
                        pltpu.VMEM((Sb, k), jnp.float32)],
    )(*args)
    return res if with_weights else (res[0] if isinstance(res, (list, tuple)) else res)


def _sc_gather(table, idx):
    R = idx.shape[0]
    C = table.shape[1]
    NW = 32
    rpw = R // NW
    ch = 128 if rpw % 128 == 0 else rpw
    nch = rpw // ch
    mesh = plsc.VectorSubcoreMesh(core_axis_name="c", subcore_axis_name="s")

    @functools.partial(
        pl.kernel,
        mesh=mesh,
        compiler_params=pltpu.CompilerParams(use_tc_tiling_on_sc=False),
        out_type=jax.ShapeDtypeStruct((R, C), jnp.float32),
        scratch_types=[
            pltpu.VMEM((ch,), jnp.int32),
            pltpu.VMEM((ch, C), jnp.float32),
            pltpu.SemaphoreType.DMA,
        ],
    )
    def k(table_hbm, idx_hbm, out_hbm, idx_v, rows_v, sem):
        wid = lax.axis_index("s") * 2 + lax.axis_index("c")
        base = wid * rpw

        def chunk(c, carry):
            r0 = base + c * ch
            pltpu.sync_copy(idx_hbm.at[pl.ds(r0, ch)], idx_v)
            pltpu.async_copy(table_hbm.at[idx_v], rows_v, sem).wait()
            pltpu.sync_copy(rows_v, out_hbm.at[pl.ds(r0, ch)])
            return carry

        lax.fori_loop(0, nch, chunk, 0)

    return k(table, idx)


def _sa_tail(gx, gf, q, wxt, b1, layers):
    Q = q.shape[0]
    c1 = wxt.shape[1]
    Qb = min(Q, 128)
    cout = layers[-1][0].shape[1]

    def body(gx_ref, *refs):
        out_ref = refs[-1]
        if gf is None:
            q_ref, wxt_ref, b1_ref = refs[:3]
            wrefs = refs[3:-1]
        else:
            gf_ref, q_ref, wxt_ref, b1_ref = refs[:4]
            wrefs = refs[4:-1]
        gx3 = gx_ref[...].reshape(Qb, _K, 16)[:, :, :3]
        rel = (gx3 - q_ref[...][:, None, :]).reshape(Qb * _K, 3)
        h = _dot(rel, wxt_ref[...]) + b1_ref[...][None, :]
        if gf is not None:
            h = h + gf_ref[...]
        h = jnp.maximum(h * _BN, 0.0)
        for li in range(len(layers)):
            wt = wrefs[2 * li][...]
            b = wrefs[2 * li + 1][...]
            h = jnp.maximum((_dot(h, wt) + b[None, :]) * _BN, 0.0)
        out_ref[...] = jnp.max(h.reshape(Qb, _K, cout), axis=1)

    in_specs = [pl.BlockSpec((Qb * _K, 16), lambda i: (i, 0))]
    args = [gx]
    if gf is not None:
        in_specs.append(pl.BlockSpec((Qb * _K, c1), lambda i: (i, 0)))
        args.append(gf)
    in_specs += [pl.BlockSpec((Qb, 3), lambda i: (i, 0)),
                 pl.BlockSpec(wxt.shape, lambda i: (0, 0)),
                 pl.BlockSpec(b1.shape, lambda i: (0,))]
    args += [q, wxt, b1]
    for (wt, b, _a) in layers:
        in_specs.append(pl.BlockSpec(wt.shape, lambda i: (0, 0)))
        in_specs.append(pl.BlockSpec(b.shape, lambda i: (0,)))
        args += [wt, b]
    return pl.pallas_call(
        body,
        grid=(Q // Qb,),
        in_specs=in_specs,
        out_specs=pl.BlockSpec((Qb, cout), lambda i: (i, 0)),
        out_shape=jax.ShapeDtypeStruct((Q, cout), jnp.float32),
    )(*args)


def _fp_tail(g, w, z1, b1, w1at, layers):
    Q3, ca = g.shape
    Q = Q3 // 3
    Qb = min(Q, 256)
    cm = w1at.shape[1]
    cout = layers[-1][0].shape[1]

    def body(g_ref, w_ref, z_ref, b1_ref, w1a_ref, *refs):
        out_ref = refs[-1]
        g3 = g_ref[...].reshape(Qb, 3, ca)
        interp = jnp.sum(g3 * w_ref[...][:, :, None], axis=1)
        h = _dot(interp, w1a_ref[...]) + z_ref[...] + b1_ref[...][None, :]
        h = jnp.maximum(h * _BN, 0.0)
        for li in range(len(layers)):
            wt = refs[2 * li][...]
            b = refs[2 * li + 1][...]
            h = _dot(h, wt) + b[None, :]
            if layers[li][2]:
                h = jnp.maximum(h * _BN, 0.0)
        out_ref[...] = h

    in_specs = [pl.BlockSpec((Qb * 3, ca), lambda i: (i, 0)),
                pl.BlockSpec((Qb, 3), lambda i: (i, 0)),
                pl.BlockSpec((Qb, cm), lambda i: (i, 0)),
                pl.BlockSpec(b1.shape, lambda i: (0,)),
                pl.BlockSpec(w1at.shape, lambda i: (0, 0))]
    args = [g, w, z1, b1, w1at]
    for (wt, b, _a) in layers:
        in_specs.append(pl.BlockSpec(wt.shape, lambda i: (0, 0)))
        in_specs.append(pl.BlockSpec(b.shape, lambda i: (0,)))
        args += [wt, b]
    return pl.pallas_call(
        body,
        grid=(Q // Qb,),
        in_specs=in_specs,
        out_specs=pl.BlockSpec((Qb, cout), lambda i: (i, 0)),
        out_shape=jax.ShapeDtypeStruct((Q, cout), jnp.float32),
    )(*args)


def _sa_global(xyz3, f, w1xt, w1ft, b1, w2t, b2):
    B, P, C = f.shape
    cmid = w1xt.shape[1]
    cout = w2t.shape[1]

    def body(x_ref, f_ref, wx_ref, wf_ref, b1_ref, w2_ref, b2_ref, out_ref):
        x = x_ref[0]
        rel = x - jnp.mean(x, axis=0, keepdims=True)
        h = _dot(rel, wx_ref[...]) + _dot(f_ref[0], wf_ref[...]) + b1_ref[...][None, :]
        h = jnp.maximum(h * _BN, 0.0)
        h = jnp.maximum((_dot(h, w2_ref[...]) + b2_ref[...][None, :]) * _BN, 0.0)
        out_ref[0, 0] = jnp.max(h, axis=0)

    return pl.pallas_call(
        body,
        grid=(B,),
        in_specs=[
            pl.BlockSpec((1, P, 3), lambda b: (b, 0, 0)),
            pl.BlockSpec((1, P, C), lambda b: (b, 0, 0)),
            pl.BlockSpec(w1xt.shape, lambda b: (0, 0)),
            pl.BlockSpec(w1ft.shape, lambda b: (0, 0)),
            pl.BlockSpec(b1.shape, lambda b: (0,)),
            pl.BlockSpec(w2t.shape, lambda b: (0, 0)),
            pl.BlockSpec(b2.shape, lambda b: (0,)),
        ],
        out_specs=pl.BlockSpec((1, 1, cout), lambda b: (b, 0, 0)),
        out_shape=jax.ShapeDtypeStruct((B, 1, cout), jnp.float32),
    )(xyz3, f, w1xt, w1ft, b1, w2t, b2)[:, 0]


def _fps_indices(B, N, npoint, level):
    skey = jax.random.key(42)
    keys = jax.random.split(jax.random.fold_in(skey, level), B)
    return jax.vmap(lambda k: jax.random.permutation(k, N)[:npoint])(keys)


def _pad16(x):
    B, N, _ = x.shape
    return jnp.concatenate([x, jnp.zeros((B, N, 13), jnp.float32)], axis=-1)


def _sa_level(level, plist, npoint, xyz, feats):
    B, N, _ = xyz.shape
    (w1, b1) = plist[0]
    c1 = w1.shape[0]
    fps = _fps_indices(B, N, npoint, level)
    fps_flat = (fps + jnp.arange(B, dtype=fps.dtype)[:, None] * N).reshape(-1)
    xyz16 = _pad16(xyz).reshape(B * N, 16)
    new_xyz = _sc_gather(xyz16, fps_flat.astype(jnp.int32))[:, :3]
    new_xyz = new_xyz.reshape(B, npoint, 3)

    idx = _knn_kernel(new_xyz, xyz, _K)
    idx_flat = idx.reshape(-1)
    gx = _sc_gather(xyz16, idx_flat)
    if feats is None:
        gf = None
    else:
        yf = _rows_mlp(feats.reshape(B * N, feats.shape[-1]),
                       [(w1[:, 3:].T, None, False)])
        gf = _sc_gather(yf, idx_flat)
    layers = [(w.T, b, True) for (w, b) in plist[1:]]
    nf = _sa_tail(gx, gf, new_xyz.reshape(B * npoint, 3), w1[:, :3].T, b1,
                  layers)
    return new_xyz, nf.reshape(B, npoint, -1)


def _fp_level(plist, xyz1, xyz2, f1, f2, extra_layers=()):
    B, N, _ = xyz1.shape
    S = xyz2.shape[1]
    (w1, b1) = plist[0]
    ca = f2.shape[-1]
    idx, w = _knn_kernel(xyz1, xyz2, 3, with_weights=True)
    z1 = _rows_mlp(f1.reshape(B * N, f1.shape[-1]),
                   [(w1[:, ca:].T, None, False)])
    g = _sc_gather(f2.reshape(B * S, ca), idx.reshape(-1))
    layers = [(wb[0].T, wb[1], True) for wb in plist[1:]] + list(extra_layers)
    out = _fp_tail(g, w.reshape(B * N, 3), z1, b1, w1[:, :ca].T, layers)
    return out.reshape(B, N, -1)


def kernel(xyz, params):
    B, N, _ = xyz.shape
    l1_xyz, l1_f = _sa_level(1, params['sa1'], _NPOINTS[0], xyz, None)
    l2_xyz, l2_f = _sa_level(2, params['sa2'], _NPOINTS[1], l1_xyz, l1_f)
    l3_xyz, l3_f = _sa_level(3, params['sa3'], _NPOINTS[2], l2_xyz, l2_f)

    (w1, b1), (w2, b2) = params['sa4']
    l4_f = _sa_global(l3_xyz, l3_f, w1[:, :3].T, w1[:, 3:].T, b1, w2.T, b2)

    P3 = l3_f.shape[1]
    cat = jnp.concatenate(
        [jnp.broadcast_to(l4_f[:, None, :], (B, P3, l4_f.shape[-1])), l3_f],
        axis=-1)
    fp4_layers = [(w.T, b, True) for (w, b) in params['fp4']]
    l3_f = _rows_mlp(cat.reshape(B * P3, cat.shape[-1]),
                     fp4_layers).reshape(B, P3, -1)

    l2_f = _fp_level(params['fp3'], l2_xyz, l3_xyz, l2_f, l3_f)
    l1_f = _fp_level(params['fp2'], l1_xyz, l2_xyz, l1_f, l2_f)

    head = params['head']
    extra = [(head[0][0].T, head[0][1], True), (head[1][0].T, head[1][1], False)]
    out = _fp_level(params['fp1'], xyz, l1_xyz, xyz, l1_f, extra_layers=extra)
    return out


# speedup vs baseline: 9.5166x; 1.0663x over previous
import functools

import jax
import jax.numpy as jnp
import numpy as np
from jax import lax
from jax.experimental import pallas as pl
from jax.experimental.pallas import tpu as pltpu
from jax.experimental.pallas import tpu_sc as plsc

_BN = 1.0 / np.sqrt(1.0 + 1e-5)
_NPOINTS = [1024, 256, 64]
_K = 32


def _dot(a, b):
    return lax.dot_general(a, b, (((a.ndim - 1,), (0,)), ((), ())),
                           preferred_element_type=jnp.float32)


def _rows_mlp(x, layers):
    layers = [(wt, jnp.zeros((wt.shape[1],), jnp.float32) if b is None else b, a)
              for (wt, b, a) in layers]
    M = x.shape[0]
    Mb = M if M <= 2048 else 2048
    cout = layers[-1][0].shape[1]

    def body(x_ref, *refs):
        out_ref = refs[-1]
        h = x_ref[...]
        for li in range(len(layers)):
            wt = refs[2 * li][...]
            b = refs[2 * li + 1][...]
            h = _dot(h, wt) + b[None, :]
            if layers[li][2]:
                h = jnp.maximum(h * _BN, 0.0)
        out_ref[...] = h

    in_specs = [pl.BlockSpec((Mb, x.shape[1]), lambda i: (i, 0))]
    args = [x]
    for (wt, b, _a) in layers:
        in_specs.append(pl.BlockSpec(wt.shape, lambda i: (0, 0)))
        in_specs.append(pl.BlockSpec(b.shape, lambda i: (0,)))
        args += [wt, b]
    return pl.pallas_call(
        body,
        grid=(M // Mb,),
        in_specs=in_specs,
        out_specs=pl.BlockSpec((Mb, cout), lambda i: (i, 0)),
        out_shape=jax.ShapeDtypeStruct((M, cout), jnp.float32),
    )(*args)


def _knn_kernel(q, pts, k, with_weights=False):
    B, S, _ = q.shape
    N = pts.shape[1]
    Sb = min(S, 512)

    def body(q_ref, p_ref, *refs):
        if with_weights:
            idx_ref, w_ref, dist_ref, dv_ref = refs
        else:
            idx_ref, dist_ref, dv_ref = refs
        b = pl.program_id(0)
        qb = q_ref[0]
        pb = p_ref[0]
        q2 = jnp.sum(qb * qb, axis=1)
        p2 = jnp.sum(pb * pb, axis=1)
        cross = lax.dot_general(qb, pb, (((1,), (1,)), ((), ())),
                                preferred_element_type=jnp.float32)
        dist_ref[...] = jnp.maximum(q2[:, None] + p2[None, :] - 2.0 * cross, 0.0)
        iota = lax.broadcasted_iota(jnp.int32, (Sb, N), 1)

        kiota = lax.broadcasted_iota(jnp.int32, (Sb, k), 1)

        def step(j, carry):
            d = dist_ref[...]
            m = jnp.min(d, axis=1)
            sel = jnp.min(jnp.where(d <= m[:, None], iota, N), axis=1)
            idx_ref[0] = jnp.where(kiota == j, (sel + b * N)[:, None],
                                   idx_ref[0])
            dist_ref[...] = jnp.where(iota == sel[:, None], jnp.inf, d)
            if with_weights:
                dv_ref[...] = jnp.where(kiota == j, m[:, None], dv_ref[...])
            return carry

        lax.fori_loop(0, k, step, 0)
        if with_weights:
            recip = 1.0 / (dv_ref[...] + 1e-8)
            w_ref[0] = recip / jnp.sum(recip, axis=1, keepdims=True)

    in_specs = [
        pl.BlockSpec((1, Sb, 3), lambda b, s: (b, s, 0)),
        pl.BlockSpec((1, N, 3), lambda b, s: (b, 0, 0)),
    ]
    args = [q, pts]
    out_specs = [pl.BlockSpec((1, Sb, k), lambda b, s: (b, s, 0))]
    out_shape = [jax.ShapeDtypeStruct((B, S, k), jnp.int32)]
    if with_weights:
        out_specs.append(pl.BlockSpec((1, Sb, k), lambda b, s: (b, s, 0)))
        out_shape.append(jax.ShapeDtypeStruct((B, S, k), jnp.float32))
    res = pl.pallas_call(
        body,
        grid=(B, S // Sb),
        in_specs=in_specs,
        out_specs=out_specs,
        out_shape=out_shape,
        scratch_shapes=[pltpu.VMEM((Sb, N), jnp.float32),
                        pltpu.VMEM((Sb, k), jnp.float32)],
    )(*args)
    return res if with_weights else (res[0] if isinstance(res, (list, tuple)) else res)


def _sc_gather(table, idx):
    R = idx.shape[0]
    C = table.shape[1]
    NW = 32
    rpw = R // NW
    ch = 128 if rpw % 128 == 0 else rpw
    nch = rpw // ch
    mesh = plsc.VectorSubcoreMesh(core_axis_name="c", subcore_axis_name="s")

    @functools.partial(
        pl.kernel,
        mesh=mesh,
        compiler_params=pltpu.CompilerParams(use_tc_tiling_on_sc=False),
        out_type=jax.ShapeDtypeStruct((R, C), jnp.float32),
        scratch_types=[
            pltpu.VMEM((ch,), jnp.int32),
            pltpu.VMEM((ch, C), jnp.float32),
            pltpu.SemaphoreType.DMA,
        ],
    )
    def k(table_hbm, idx_hbm, out_hbm, idx_v, rows_v, sem):
        wid = lax.axis_index("s") * 2 + lax.axis_index("c")
        base = wid * rpw

        def chunk(c, carry):
            r0 = base + c * ch
            pltpu.sync_copy(idx_hbm.at[pl.ds(r0, ch)], idx_v)
            pltpu.async_copy(table_hbm.at[idx_v], rows_v, sem).wait()
            pltpu.sync_copy(rows_v, out_hbm.at[pl.ds(r0, ch)])
            return carry

        lax.fori_loop(0, nch, chunk, 0)

    return k(table, idx)


def _sc_gather2(t1, t2, idx):
    R = idx.shape[0]
    C1 = t1.shape[1]
    C2 = t2.shape[1]
    NW = 32
    rpw = R // NW
    ch = 128 if rpw % 128 == 0 else rpw
    nch = rpw // ch
    mesh = plsc.VectorSubcoreMesh(core_axis_name="c", subcore_axis_name="s")

    @functools.partial(
        pl.kernel,
        mesh=mesh,
        compiler_params=pltpu.CompilerParams(use_tc_tiling_on_sc=False),
        out_type=(jax.ShapeDtypeStruct((R, C1), jnp.float32),
                  jax.ShapeDtypeStruct((R, C2), jnp.float32)),
        scratch_types=[
            pltpu.VMEM((ch,), jnp.int32),
            pltpu.VMEM((ch, C1), jnp.float32),
            pltpu.VMEM((ch, C2), jnp.float32),
            pltpu.SemaphoreType.DMA,
            pltpu.SemaphoreType.DMA,
        ],
    )
    def k(t1_hbm, t2_hbm, idx_hbm, o1_hbm, o2_hbm, idx_v, r1_v, r2_v, s1, s2):
        wid = lax.axis_index("s") * 2 + lax.axis_index("c")
        base = wid * rpw

        def chunk(c, carry):
            r0 = base + c * ch
            pltpu.sync_copy(idx_hbm.at[pl.ds(r0, ch)], idx_v)
            d1 = pltpu.async_copy(t1_hbm.at[idx_v], r1_v, s1)
            d2 = pltpu.async_copy(t2_hbm.at[idx_v], r2_v, s2)
            d1.wait()
            d2.wait()
            pltpu.sync_copy(r1_v, o1_hbm.at[pl.ds(r0, ch)])
            pltpu.sync_copy(r2_v, o2_hbm.at[pl.ds(r0, ch)])
            return carry

        lax.fori_loop(0, nch, chunk, 0)

    return k(t1, t2, idx)


def _sa_tail(gx, gf, q, wxt, b1, layers):
    Q = q.shape[0]
    c1 = wxt.shape[1]
    Qb = min(Q, 256)
    cout = layers[-1][0].shape[1]

    def body(gx_ref, *refs):
        out_ref = refs[-1]
        if gf is None:
            q_ref, wxt_ref, b1_ref = refs[:3]
            wrefs = refs[3:-1]
        else:
            gf_ref, q_ref, wxt_ref, b1_ref = refs[:4]
            wrefs = refs[4:-1]
        gx3 = gx_ref[...].reshape(Qb, _K, 16)[:, :, :3]
        rel = (gx3 - q_ref[...][:, None, :]).reshape(Qb * _K, 3)
        h = _dot(rel, wxt_ref[...]) + b1_ref[...][None, :]
        if gf is not None:
            h = h + gf_ref[...]
        h = jnp.maximum(h * _BN, 0.0)
        for li in range(len(layers)):
            wt = wrefs[2 * li][...]
            b = wrefs[2 * li + 1][...]
            h = jnp.maximum((_dot(h, wt) + b[None, :]) * _BN, 0.0)
        out_ref[...] = jnp.max(h.reshape(Qb, _K, cout), axis=1)

    in_specs = [pl.BlockSpec((Qb * _K, 16), lambda i: (i, 0))]
    args = [gx]
    if gf is not None:
        in_specs.append(pl.BlockSpec((Qb * _K, c1), lambda i: (i, 0)))
        args.append(gf)
    in_specs += [pl.BlockSpec((Qb, 3), lambda i: (i, 0)),
                 pl.BlockSpec(wxt.shape, lambda i: (0, 0)),
                 pl.BlockSpec(b1.shape, lambda i: (0,))]
    args += [q, wxt, b1]
    for (wt, b, _a) in layers:
        in_specs.append(pl.BlockSpec(wt.shape, lambda i: (0, 0)))
        in_specs.append(pl.BlockSpec(b.shape, lambda i: (0,)))
        args += [wt, b]
    return pl.pallas_call(
        body,
        grid=(Q // Qb,),
        in_specs=in_specs,
        out_specs=pl.BlockSpec((Qb, cout), lambda i: (i, 0)),
        out_shape=jax.ShapeDtypeStruct((Q, cout), jnp.float32),
    )(*args)


def _fp_tail(g, w, z1, b1, w1at, layers):
    Q3, ca = g.shape
    Q = Q3 // 3
    Qb = min(Q, 512)
    cm = w1at.shape[1]
    cout = layers[-1][0].shape[1]

    def body(g_ref, w_ref, z_ref, b1_ref, w1a_ref, *refs):
        out_ref = refs[-1]
        g3 = g_ref[...].reshape(Qb, 3, ca)
        interp = jnp.sum(g3 * w_ref[...][:, :, None], axis=1)
        h = _dot(interp, w1a_ref[...]) + z_ref[...] + b1_ref[...][None, :]
        h = jnp.maximum(h * _BN, 0.0)
        for li in range(len(layers)):
            wt = refs[2 * li][...]
            b = refs[2 * li + 1][...]
            h = _dot(h, wt) + b[None, :]
            if layers[li][2]:
                h = jnp.maximum(h * _BN, 0.0)
        out_ref[...] = h

    in_specs = [pl.BlockSpec((Qb * 3, ca), lambda i: (i, 0)),
                pl.BlockSpec((Qb, 3), lambda i: (i, 0)),
                pl.BlockSpec((Qb, cm), lambda i: (i, 0)),
                pl.BlockSpec(b1.shape, lambda i: (0,)),
                pl.BlockSpec(w1at.shape, lambda i: (0, 0))]
    args = [g, w, z1, b1, w1at]
    for (wt, b, _a) in layers:
        in_specs.append(pl.BlockSpec(wt.shape, lambda i: (0, 0)))
        in_specs.append(pl.BlockSpec(b.shape, lambda i: (0,)))
        args += [wt, b]
    return pl.pallas_call(
        body,
        grid=(Q // Qb,),
        in_specs=in_specs,
        out_specs=pl.BlockSpec((Qb, cout), lambda i: (i, 0)),
        out_shape=jax.ShapeDtypeStruct((Q, cout), jnp.float32),
    )(*args)


def _sa_global(xyz3, f, w1xt, w1ft, b1, w2t, b2):
    B, P, C = f.shape
    cmid = w1xt.shape[1]
    cout = w2t.shape[1]

    def body(x_ref, f_ref, wx_ref, wf_ref, b1_ref, w2_ref, b2_ref, out_ref):
        x = x_ref[0]
        rel = x - jnp.mean(x, axis=0, keepdims=True)
        h = _dot(rel, wx_ref[...]) + _dot(f_ref[0], wf_ref[...]) + b1_ref[...][None, :]
        h = jnp.maximum(h * _BN, 0.0)
        h = jnp.maximum((_dot(h, w2_ref[...]) + b2_ref[...][None, :]) * _BN, 0.0)
        out_ref[0, 0] = jnp.max(h, axis=0)

    return pl.pallas_call(
        body,
        grid=(B,),
        in_specs=[
            pl.BlockSpec((1, P, 3), lambda b: (b, 0, 0)),
            pl.BlockSpec((1, P, C), lambda b: (b, 0, 0)),
            pl.BlockSpec(w1xt.shape, lambda b: (0, 0)),
            pl.BlockSpec(w1ft.shape, lambda b: (0, 0)),
            pl.BlockSpec(b1.shape, lambda b: (0,)),
            pl.BlockSpec(w2t.shape, lambda b: (0, 0)),
            pl.BlockSpec(b2.shape, lambda b: (0,)),
        ],
        out_specs=pl.BlockSpec((1, 1, cout), lambda b: (b, 0, 0)),
        out_shape=jax.ShapeDtypeStruct((B, 1, cout), jnp.float32),
    )(xyz3, f, w1xt, w1ft, b1, w2t, b2)[:, 0]


def _fps_indices(B, N, npoint, level):
    skey = jax.random.key(42)
    keys = jax.random.split(jax.random.fold_in(skey, level), B)
    return jax.vmap(lambda k: jax.random.permutation(k, N)[:npoint])(keys)


def _pad16(x):
    B, N, _ = x.shape
    return jnp.concatenate([x, jnp.zeros((B, N, 13), jnp.float32)], axis=-1)


def _sa_level(level, plist, npoint, xyz, feats):
    B, N, _ = xyz.shape
    (w1, b1) = plist[0]
    c1 = w1.shape[0]
    fps = _fps_indices(B, N, npoint, level)
    fps_flat = (fps + jnp.arange(B, dtype=fps.dtype)[:, None] * N).reshape(-1)
    xyz16 = _pad16(xyz).reshape(B * N, 16)
    new_xyz = _sc_gather(xyz16, fps_flat.astype(jnp.int32))[:, :3]
    new_xyz = new_xyz.reshape(B, npoint, 3)

    idx = _knn_kernel(new_xyz, xyz, _K)
    idx_flat = idx.reshape(-1)
    if feats is None:
        gx = _sc_gather(xyz16, idx_flat)
        gf = None
    else:
        yf = _rows_mlp(feats.reshape(B * N, feats.shape[-1]),
                       [(w1[:, 3:].T, None, False)])
        gx, gf = _sc_gather2(xyz16, yf, idx_flat)
    layers = [(w.T, b, True) for (w, b) in plist[1:]]
    nf = _sa_tail(gx, gf, new_xyz.reshape(B * npoint, 3), w1[:, :3].T, b1,
                  layers)
    return new_xyz, nf.reshape(B, npoint, -1)


def _fp_level(plist, xyz1, xyz2, f1, f2, extra_layers=()):
    B, N, _ = xyz1.shape
    S = xyz2.shape[1]
    (w1, b1) = plist[0]
    ca = f2.shape[-1]
    idx, w = _knn_kernel(xyz1, xyz2, 3, with_weights=True)
    z1 = _rows_mlp(f1.reshape(B * N, f1.shape[-1]),
                   [(w1[:, ca:].T, None, False)])
    g = _sc_gather(f2.reshape(B * S, ca), idx.reshape(-1))
    layers = [(wb[0].T, wb[1], True) for wb in plist[1:]] + list(extra_layers)
    out = _fp_tail(g, w.reshape(B * N, 3), z1, b1, w1[:, :ca].T, layers)
    return out.reshape(B, N, -1)


def kernel(xyz, params):
    B, N, _ = xyz.shape
    l1_xyz, l1_f = _sa_level(1, params['sa1'], _NPOINTS[0], xyz, None)
    l2_xyz, l2_f = _sa_level(2, params['sa2'], _NPOINTS[1], l1_xyz, l1_f)
    l3_xyz, l3_f = _sa_level(3, params['sa3'], _NPOINTS[2], l2_xyz, l2_f)

    (w1, b1), (w2, b2) = params['sa4']
    l4_f = _sa_global(l3_xyz, l3_f, w1[:, :3].T, w1[:, 3:].T, b1, w2.T, b2)

    P3 = l3_f.shape[1]
    cat = jnp.concatenate(
        [jnp.broadcast_to(l4_f[:, None, :], (B, P3, l4_f.shape[-1])), l3_f],
        axis=-1)
    fp4_layers = [(w.T, b, True) for (w, b) in params['fp4']]
    l3_f = _rows_mlp(cat.reshape(B * P3, cat.shape[-1]),
                     fp4_layers).reshape(B, P3, -1)

    l2_f = _fp_level(params['fp3'], l2_xyz, l3_xyz, l2_f, l3_f)
    l1_f = _fp_level(params['fp2'], l1_xyz, l2_xyz, l1_f, l2_f)

    head = params['head']
    extra = [(head[0][0].T, head[0][1], True), (head[1][0].T, head[1][1], False)]
    out = _fp_level(params['fp1'], xyz, l1_xyz, xyz, l1_f, extra_layers=extra)
    return out
